# baseline (speedup 1.0000x reference)
import jax
import jax.numpy as jnp
from jax import lax
from jax.experimental import pallas as pl
from jax.experimental.pallas import tpu as pltpu
from jax.experimental.pallas import tpu_sc as plsc

B = 4096
L = 20
V = 1000000
D = 32
H1 = 128
H2 = 64

NC = 2
NS = 16
NW = NC * NS
SPW = B // NW
RPW = SPW * L
CH = 128
NCHUNK = RPW // CH
NB = 4
LAG = 2
PACK = 128 // D
AR = SPW * PACK
NPH = 4
SPH = SPW // NPH
HR = AR // NPH
CPH = NCHUNK // NPH
NBUF = 3
BUFR = NS * HR

TLANE = 4096
TROW = TLANE // PACK
TGRID = -(-V // TLANE)
VP = TGRID * TROW


def _repack_body(xq_ref, xc_ref, oq_ref, oc_ref):
    ids = lax.broadcasted_iota(jnp.int32, (D, PACK * D), 0)
    ils = lax.broadcasted_iota(jnp.int32, (D, PACK * D), 1)

    def pack(x_ref):
        acc = None
        for a in range(PACK):
            e_a = (ils == a * D + ids).astype(jnp.float32)
            y = lax.dot_general(
                x_ref[:, a * TROW:(a + 1) * TROW], e_a,
                (((0,), (0,)), ((), ())),
                precision=lax.Precision.HIGHEST,
                preferred_element_type=jnp.float32)
            acc = y if acc is None else acc + y
        return acc

    oq_ref[...] = pack(xq_ref)
    oc_ref[...] = pack(xc_ref)


def _repack_tc(tqT, tcT):
    return pl.pallas_call(
        _repack_body,
        grid=(TGRID,),
        in_specs=[
            pl.BlockSpec((D, TLANE), lambda i: (0, i)),
            pl.BlockSpec((D, TLANE), lambda i: (0, i)),
        ],
        out_specs=[
            pl.BlockSpec((TROW, PACK * D), lambda i: (i, 0)),
            pl.BlockSpec((TROW, PACK * D), lambda i: (i, 0)),
        ],
        out_shape=[
            jax.ShapeDtypeStruct((VP, PACK * D), jnp.float32),
            jax.ShapeDtypeStruct((VP, PACK * D), jnp.float32),
        ],
    )(tqT, tcT)


def _pool_body(tq_hbm, tc_hbm, qpidx_hbm, cpidx_hbm, sidx_hbm, zeros_hbm,
               accq_hbm, accc_hbm,
               pidx_v, sidxq_v, sidxc_v, rows_v, acc_sh, gsem, ssem):
    cid = lax.axis_index("c")
    sid = lax.axis_index("s")
    wid = sid * NC + cid
    base_i = wid * RPW
    base_a = sid * HR

    pltpu.sync_copy(qpidx_hbm.at[pl.ds(base_i, RPW)], pidx_v.at[0])
    pltpu.sync_copy(cpidx_hbm.at[pl.ds(base_i, RPW)], pidx_v.at[1])
    pltpu.sync_copy(sidx_hbm.at[wid], sidxq_v)
    pltpu.sync_copy(sidx_hbm.at[NW + wid], sidxc_v)
    pltpu.sync_copy(zeros_hbm, acc_sh.at[pl.ds(0 * BUFR + base_a, HR)])
    pltpu.sync_copy(zeros_hbm, acc_sh.at[pl.ds(1 * BUFR + base_a, HR)])

    tbls = (tq_hbm, tc_hbm)
    sidxs = (sidxq_v, sidxc_v)
    outs = (accq_hbm, accc_hbm)
    NWORK = 2 * NCHUNK

    def gather(k):
        tower, j = divmod(k, NCHUNK)
        return pltpu.async_copy(
            tbls[tower].at[pidx_v.at[tower, pl.ds(j * CH, CH)]],
            rows_v.at[k % NB], gsem)

    NPHASES = 2 * NPH

    def flush(ph):
        tower, quarter = divmod(ph, NPH)
        pltpu.sync_copy(
            acc_sh.at[pl.ds((ph % NBUF) * BUFR + base_a, HR)],
            outs[tower].at[pl.ds(wid * AR + quarter * HR, HR)])

    gd = [None] * NWORK
    sd = [None] * NWORK
    waited = [False] * NWORK
    for k in range(NWORK + LAG):
        if k < NWORK:
            if k >= NB and not waited[k - NB]:
                sd[k - NB].wait()
                waited[k - NB] = True
            gd[k] = gather(k)
        ks = k - LAG
        if 0 <= ks < NWORK:
            if ks % CPH == 0 and ks > 0:
                p = ks // CPH
                for i in range(ks):
                    if not waited[i]:
                        sd[i].wait()
                        waited[i] = True
                if p >= 2:
                    flush(p - 2)
                if p + 1 < NPHASES:
                    pltpu.sync_copy(
                        zeros_hbm,
                        acc_sh.at[pl.ds(((p + 1) % NBUF) * BUFR + base_a,
                                        HR)])
            tower, j = divmod(ks, NCHUNK)
            gd[ks].wait()
            if ks > 0 and not waited[ks - 1]:
                sd[ks - 1].wait()
                waited[ks - 1] = True
            sd[ks] = pltpu.async_copy(
                rows_v.at[ks % NB], acc_sh.at[sidxs[tower].at[j]], ssem,
                add=True)
    for i in range(NWORK):
        if not waited[i]:
            sd[i].wait()
    flush(NPHASES - 2)
    flush(NPHASES - 1)


def _pooled_sc(tq2, tc2, qpidx, cpidx, sidx, zeros):
    mesh = plsc.VectorSubcoreMesh(core_axis_name="c", subcore_axis_name="s")
    return pl.kernel(
        _pool_body,
        out_type=(
            jax.ShapeDtypeStruct((B * PACK, PACK * D), jnp.float32),
            jax.ShapeDtypeStruct((B * PACK, PACK * D), jnp.float32),
        ),
        mesh=mesh,
        scratch_types=[
            pltpu.VMEM((2, RPW), jnp.int32),
            pltpu.VMEM((NCHUNK, CH), jnp.int32),
            pltpu.VMEM((NCHUNK, CH), jnp.int32),
            pltpu.VMEM((NB, CH, PACK * D), jnp.float32),
            pltpu.VMEM_SHARED((NBUF * BUFR, PACK * D), jnp.float32),
            pltpu.SemaphoreType.DMA,
            pltpu.SemaphoreType.DMA,
        ],
        compiler_params=pltpu.CompilerParams(use_tc_tiling_on_sc=True),
    )(tq2, tc2, qpidx, cpidx, sidx, zeros)


def _mlp_body(aq_ref, ac_ref, wq1_ref, bq1_ref, wq2_ref, bq2_ref,
              wc1_ref, bc1_ref, wc2_ref, bc2_ref, oq_ref, oc_ref):
    blk = aq_ref.shape[0] // PACK

    def pool(a_ref):
        a = a_ref[...].reshape(blk, PACK, PACK * D)
        x = a[:, 0, 0:D]
        for r in range(1, PACK):
            x = x + a[:, r, D * r: D * (r + 1)]
        return x

    xq = pool(aq_ref)
    hq = jnp.dot(xq, wq1_ref[...], preferred_element_type=jnp.float32)
    hq = jnp.maximum(hq + bq1_ref[...], 0.0)
    oq = jnp.dot(hq, wq2_ref[...], preferred_element_type=jnp.float32)
    oq_ref[...] = jnp.maximum(oq + bq2_ref[...], 0.0)

    xc = pool(ac_ref)
    hc = jnp.dot(xc, wc1_ref[...], preferred_element_type=jnp.float32)
    hc = jnp.maximum(hc + bc1_ref[...], 0.0)
    oc = jnp.dot(hc, wc2_ref[...], preferred_element_type=jnp.float32)
    oc_ref[...] = jnp.maximum(oc + bc2_ref[...], 0.0)


def _mlp_tc(accq, accc, Wq1, bq1, Wq2, bq2, Wc1, bc1, Wc2, bc2):
    BLK = 512
    grid = (B // BLK,)
    full = lambda r, c: pl.BlockSpec((r, c), lambda i: (0, 0))
    return pl.pallas_call(
        _mlp_body,
        grid=grid,
        in_specs=[
            pl.BlockSpec((BLK * PACK, PACK * D), lambda i: (i, 0)),
            pl.BlockSpec((BLK * PACK, PACK * D), lambda i: (i, 0)),
            full(D, H1), full(1, H1), full(H1, H2), full(1, H2),
            full(D, H1), full(1, H1), full(H1, H2), full(1, H2),
        ],
        out_specs=[
            pl.BlockSpec((BLK, H2), lambda i: (i, 0)),
            pl.BlockSpec((BLK, H2), lambda i: (i, 0)),
        ],
        out_shape=[
            jax.ShapeDtypeStruct((B, H2), jnp.float32),
            jax.ShapeDtypeStruct((B, H2), jnp.float32),
        ],
    )(accq, accc, Wq1, bq1, Wq2, bq2, Wc1, bc1, Wc2, bc2)


def kernel(query_indices, candidate_indices, table_q, table_c,
           Wq1, bq1, Wq2, bq2, Wc1, bc1, Wc2, bc2):
    qidx_flat = query_indices.astype(jnp.int32).reshape(B * L)
    cidx_flat = candidate_indices.astype(jnp.int32).reshape(B * L)

    tq2, tc2 = _repack_tc(table_q.T, table_c.T)

    qpidx = (qidx_flat // TLANE) * TROW + qidx_flat % TROW
    cpidx = (cidx_flat // TLANE) * TROW + cidx_flat % TROW
    pos = jnp.arange(B * L, dtype=jnp.int32)
    chunk_in_tower = (pos % RPW) // CH
    base = PACK * ((pos // L) % SPH) + (pos // RPW // NC) * HR
    ph_q = chunk_in_tower // CPH
    ph_c = NPH + ph_q
    sidx = jnp.stack([
        (base + (ph_q % NBUF) * BUFR
         + (qidx_flat % TLANE) // TROW).reshape(NW, NCHUNK, CH),
        (base + (ph_c % NBUF) * BUFR
         + (cidx_flat % TLANE) // TROW).reshape(NW, NCHUNK, CH),
    ]).reshape(2 * NW, NCHUNK, CH)
    zeros = jnp.zeros((HR, PACK * D), jnp.float32)

    accq, accc = _pooled_sc(tq2, tc2, qpidx, cpidx, sidx, zeros)

    q, c = _mlp_tc(accq, accc,
                   Wq1, bq1[None, :], Wq2, bq2[None, :],
                   Wc1, bc1[None, :], Wc2, bc2[None, :])
    return q, c


# speedup vs baseline: 1.6766x; 1.6766x over previous
import jax
import jax.numpy as jnp
from jax import lax
from jax.experimental import pallas as pl
from jax.experimental.pallas import tpu as pltpu
from jax.experimental.pallas import tpu_sc as plsc

B = 4096
L = 20
V = 1000000
D = 32
H1 = 128
H2 = 64

NC = 2
NS = 16
NW = NC * NS
SPW = B // NW
RPW = SPW * L
CH = 128
NCHUNK = RPW // CH
NB = 4
LAG = 2
PACK = 128 // D
AR = SPW * PACK
NPH = 4
SPH = SPW // NPH
HR = AR // NPH
CPH = NCHUNK // NPH
NBUF = 3
BUFR = NS * HR

TLANE = 4096
TROW = TLANE // PACK
TGRID = -(-V // TLANE)
VP = TGRID * TROW


def _repack_body(xq_ref, xc_ref, oq_ref, oc_ref):
    ids = lax.broadcasted_iota(jnp.int32, (D, PACK * D), 0)
    ils = lax.broadcasted_iota(jnp.int32, (D, PACK * D), 1)

    def pack(x_ref):
        acc = None
        for a in range(PACK):
            e_a = (ils == a * D + ids).astype(jnp.float32)
            xa = x_ref[:, a * TROW:(a + 1) * TROW]
            hi = xa.astype(jnp.bfloat16).astype(jnp.float32)
            lo = xa - hi
            dims = (((0,), (0,)), ((), ()))
            y = (lax.dot_general(hi, e_a, dims,
                                 preferred_element_type=jnp.float32)
                 + lax.dot_general(lo, e_a, dims,
                                   preferred_element_type=jnp.float32))
            acc = y if acc is None else acc + y
        return acc

    oq_ref[...] = pack(xq_ref)
    oc_ref[...] = pack(xc_ref)


def _repack_tc(tqT, tcT):
    return pl.pallas_call(
        _repack_body,
        grid=(TGRID,),
        in_specs=[
            pl.BlockSpec((D, TLANE), lambda i: (0, i)),
            pl.BlockSpec((D, TLANE), lambda i: (0, i)),
        ],
        out_specs=[
            pl.BlockSpec((TROW, PACK * D), lambda i: (i, 0)),
            pl.BlockSpec((TROW, PACK * D), lambda i: (i, 0)),
        ],
        out_shape=[
            jax.ShapeDtypeStruct((VP, PACK * D), jnp.float32),
            jax.ShapeDtypeStruct((VP, PACK * D), jnp.float32),
        ],
    )(tqT, tcT)


def _pool_body(tq_hbm, tc_hbm, qpidx_hbm, cpidx_hbm, sidx_hbm, zeros_hbm,
               accq_hbm, accc_hbm,
               pidx_v, sidxq_v, sidxc_v, rows_v, acc_sh, gsem, ssem):
    cid = lax.axis_index("c")
    sid = lax.axis_index("s")
    wid = sid * NC + cid
    base_i = wid * RPW
    base_a = sid * HR

    pltpu.sync_copy(qpidx_hbm.at[pl.ds(base_i, RPW)], pidx_v.at[0])
    pltpu.sync_copy(cpidx_hbm.at[pl.ds(base_i, RPW)], pidx_v.at[1])
    pltpu.sync_copy(sidx_hbm.at[wid], sidxq_v)
    pltpu.sync_copy(sidx_hbm.at[NW + wid], sidxc_v)
    pltpu.sync_copy(zeros_hbm, acc_sh.at[pl.ds(0 * BUFR + base_a, HR)])
    pltpu.sync_copy(zeros_hbm, acc_sh.at[pl.ds(1 * BUFR + base_a, HR)])

    tbls = (tq_hbm, tc_hbm)
    sidxs = (sidxq_v, sidxc_v)
    outs = (accq_hbm, accc_hbm)
    NWORK = 2 * NCHUNK

    def gather(k):
        tower, j = divmod(k, NCHUNK)
        return pltpu.async_copy(
            tbls[tower].at[pidx_v.at[tower, pl.ds(j * CH, CH)]],
            rows_v.at[k % NB], gsem)

    NPHASES = 2 * NPH

    def flush(ph):
        tower, quarter = divmod(ph, NPH)
        pltpu.sync_copy(
            acc_sh.at[pl.ds((ph % NBUF) * BUFR + base_a, HR)],
            outs[tower].at[pl.ds(wid * AR + quarter * HR, HR)])

    gd = [None] * NWORK
    sd = [None] * NWORK
    waited = [False] * NWORK
    for k in range(NWORK + LAG):
        if k < NWORK:
            if k >= NB and not waited[k - NB]:
                sd[k - NB].wait()
                waited[k - NB] = True
            gd[k] = gather(k)
        ks = k - LAG
        if 0 <= ks < NWORK:
            if ks % CPH == 0 and ks > 0:
                p = ks // CPH
                for i in range(ks):
                    if not waited[i]:
                        sd[i].wait()
                        waited[i] = True
                if p >= 2:
                    flush(p - 2)
                if p + 1 < NPHASES:
                    pltpu.sync_copy(
                        zeros_hbm,
                        acc_sh.at[pl.ds(((p + 1) % NBUF) * BUFR + base_a,
                                        HR)])
            tower, j = divmod(ks, NCHUNK)
            gd[ks].wait()
            if ks > 0 and not waited[ks - 1]:
                sd[ks - 1].wait()
                waited[ks - 1] = True
            sd[ks] = pltpu.async_copy(
                rows_v.at[ks % NB], acc_sh.at[sidxs[tower].at[j]], ssem,
                add=True)
    for i in range(NWORK):
        if not waited[i]:
            sd[i].wait()
    flush(NPHASES - 2)
    flush(NPHASES - 1)


def _pooled_sc(tq2, tc2, qpidx, cpidx, sidx, zeros):
    mesh = plsc.VectorSubcoreMesh(core_axis_name="c", subcore_axis_name="s")
    return pl.kernel(
        _pool_body,
        out_type=(
            jax.ShapeDtypeStruct((B * PACK, PACK * D), jnp.float32),
            jax.ShapeDtypeStruct((B * PACK, PACK * D), jnp.float32),
        ),
        mesh=mesh,
        scratch_types=[
            pltpu.VMEM((2, RPW), jnp.int32),
            pltpu.VMEM((NCHUNK, CH), jnp.int32),
            pltpu.VMEM((NCHUNK, CH), jnp.int32),
            pltpu.VMEM((NB, CH, PACK * D), jnp.float32),
            pltpu.VMEM_SHARED((NBUF * BUFR, PACK * D), jnp.float32),
            pltpu.SemaphoreType.DMA,
            pltpu.SemaphoreType.DMA,
        ],
        compiler_params=pltpu.CompilerParams(use_tc_tiling_on_sc=True),
    )(tq2, tc2, qpidx, cpidx, sidx, zeros)


def _mlp_body(aq_ref, ac_ref, wq1_ref, bq1_ref, wq2_ref, bq2_ref,
              wc1_ref, bc1_ref, wc2_ref, bc2_ref, oq_ref, oc_ref):
    blk = aq_ref.shape[0] // PACK

    def pool(a_ref):
        a = a_ref[...].reshape(blk, PACK, PACK * D)
        x = a[:, 0, 0:D]
        for r in range(1, PACK):
            x = x + a[:, r, D * r: D * (r + 1)]
        return x

    xq = pool(aq_ref)
    hq = jnp.dot(xq, wq1_ref[...], preferred_element_type=jnp.float32)
    hq = jnp.maximum(hq + bq1_ref[...], 0.0)
    oq = jnp.dot(hq, wq2_ref[...], preferred_element_type=jnp.float32)
    oq_ref[...] = jnp.maximum(oq + bq2_ref[...], 0.0)

    xc = pool(ac_ref)
    hc = jnp.dot(xc, wc1_ref[...], preferred_element_type=jnp.float32)
    hc = jnp.maximum(hc + bc1_ref[...], 0.0)
    oc = jnp.dot(hc, wc2_ref[...], preferred_element_type=jnp.float32)
    oc_ref[...] = jnp.maximum(oc + bc2_ref[...], 0.0)


def _mlp_tc(accq, accc, Wq1, bq1, Wq2, bq2, Wc1, bc1, Wc2, bc2):
    BLK = 512
    grid = (B // BLK,)
    full = lambda r, c: pl.BlockSpec((r, c), lambda i: (0, 0))
    return pl.pallas_call(
        _mlp_body,
        grid=grid,
        in_specs=[
            pl.BlockSpec((BLK * PACK, PACK * D), lambda i: (i, 0)),
            pl.BlockSpec((BLK * PACK, PACK * D), lambda i: (i, 0)),
            full(D, H1), full(1, H1), full(H1, H2), full(1, H2),
            full(D, H1), full(1, H1), full(H1, H2), full(1, H2),
        ],
        out_specs=[
            pl.BlockSpec((BLK, H2), lambda i: (i, 0)),
            pl.BlockSpec((BLK, H2), lambda i: (i, 0)),
        ],
        out_shape=[
            jax.ShapeDtypeStruct((B, H2), jnp.float32),
            jax.ShapeDtypeStruct((B, H2), jnp.float32),
        ],
    )(accq, accc, Wq1, bq1, Wq2, bq2, Wc1, bc1, Wc2, bc2)


def kernel(query_indices, candidate_indices, table_q, table_c,
           Wq1, bq1, Wq2, bq2, Wc1, bc1, Wc2, bc2):
    qidx_flat = query_indices.astype(jnp.int32).reshape(B * L)
    cidx_flat = candidate_indices.astype(jnp.int32).reshape(B * L)

    tq2, tc2 = _repack_tc(table_q.T, table_c.T)

    qpidx = (qidx_flat // TLANE) * TROW + qidx_flat % TROW
    cpidx = (cidx_flat // TLANE) * TROW + cidx_flat % TROW
    pos = jnp.arange(B * L, dtype=jnp.int32)
    chunk_in_tower = (pos % RPW) // CH
    base = PACK * ((pos // L) % SPH) + (pos // RPW // NC) * HR
    ph_q = chunk_in_tower // CPH
    ph_c = NPH + ph_q
    sidx = jnp.stack([
        (base + (ph_q % NBUF) * BUFR
         + (qidx_flat % TLANE) // TROW).reshape(NW, NCHUNK, CH),
        (base + (ph_c % NBUF) * BUFR
         + (cidx_flat % TLANE) // TROW).reshape(NW, NCHUNK, CH),
    ]).reshape(2 * NW, NCHUNK, CH)
    zeros = jnp.zeros((HR, PACK * D), jnp.float32)

    accq, accc = _pooled_sc(tq2, tc2, qpidx, cpidx, sidx, zeros)

    q, c = _mlp_tc(accq, accc,
                   Wq1, bq1[None, :], Wq2, bq2[None, :],
                   Wc1, bc1[None, :], Wc2, bc2[None, :])
    return q, c


# speedup vs baseline: 2.4903x; 1.4853x over previous
import jax
import jax.numpy as jnp
from jax import lax
from jax.experimental import pallas as pl
from jax.experimental.pallas import tpu as pltpu
from jax.experimental.pallas import tpu_sc as plsc

B = 4096
L = 20
V = 1000000
D = 32
H1 = 128
H2 = 64

NC = 2
NS = 16
NW = NC * NS
SPW = B // NW
RPW = SPW * L
CH = 128
NCHUNK = RPW // CH
NB = 4
LAG = 2
PACK = 128 // D
AR = SPW * PACK
NPH = 4
SPH = SPW // NPH
HR = AR // NPH
CPH = NCHUNK // NPH
NBUF = 3
BUFR = NS * HR

TLANE = 4096
TROW = TLANE // PACK
TGRID = -(-V // TLANE)
VP = TGRID * TROW


def _repack_body(xq_ref, xc_ref, e_ref, oq_ref, oc_ref):
    def pack(x_ref):
        xa = x_ref[...]
        hi = xa.astype(jnp.bfloat16).astype(jnp.float32)
        lo = xa - hi
        X = jnp.concatenate(
            [hi[:, a * TROW:(a + 1) * TROW] for a in range(PACK)]
            + [lo[:, a * TROW:(a + 1) * TROW] for a in range(PACK)], axis=0)
        return lax.dot_general(X, e_ref[...], (((0,), (0,)), ((), ())),
                               preferred_element_type=jnp.float32)

    oq_ref[...] = pack(xq_ref)
    oc_ref[...] = pack(xc_ref)


def _repack_tc(tqT, tcT, emat):
    return pl.pallas_call(
        _repack_body,
        grid=(TGRID,),
        in_specs=[
            pl.BlockSpec((D, TLANE), lambda i: (0, i)),
            pl.BlockSpec((D, TLANE), lambda i: (0, i)),
            pl.BlockSpec((2 * PACK * D, PACK * D), lambda i: (0, 0)),
        ],
        out_specs=[
            pl.BlockSpec((TROW, PACK * D), lambda i: (i, 0)),
            pl.BlockSpec((TROW, PACK * D), lambda i: (i, 0)),
        ],
        out_shape=[
            jax.ShapeDtypeStruct((VP, PACK * D), jnp.float32),
            jax.ShapeDtypeStruct((VP, PACK * D), jnp.float32),
        ],
    )(tqT, tcT, emat)


def _pool_body(tq_hbm, tc_hbm, qpidx_hbm, cpidx_hbm, sidx_hbm, zeros_hbm,
               accq_hbm, accc_hbm,
               pidx_v, sidxq_v, sidxc_v, rows_v, acc_sh, gsem, ssem):
    cid = lax.axis_index("c")
    sid = lax.axis_index("s")
    wid = sid * NC + cid
    base_i = wid * RPW
    base_a = sid * HR

    pltpu.sync_copy(qpidx_hbm.at[pl.ds(base_i, RPW)], pidx_v.at[0])
    pltpu.sync_copy(cpidx_hbm.at[pl.ds(base_i, RPW)], pidx_v.at[1])
    pltpu.sync_copy(sidx_hbm.at[wid], sidxq_v)
    pltpu.sync_copy(sidx_hbm.at[NW + wid], sidxc_v)
    pltpu.sync_copy(zeros_hbm, acc_sh.at[pl.ds(0 * BUFR + base_a, HR)])
    pltpu.sync_copy(zeros_hbm, acc_sh.at[pl.ds(1 * BUFR + base_a, HR)])

    tbls = (tq_hbm, tc_hbm)
    sidxs = (sidxq_v, sidxc_v)
    outs = (accq_hbm, accc_hbm)
    NWORK = 2 * NCHUNK

    def gather(k):
        tower, j = divmod(k, NCHUNK)
        return pltpu.async_copy(
            tbls[tower].at[pidx_v.at[tower, pl.ds(j * CH, CH)]],
            rows_v.at[k % NB], gsem)

    NPHASES = 2 * NPH

    def flush(ph):
        tower, quarter = divmod(ph, NPH)
        pltpu.sync_copy(
            acc_sh.at[pl.ds((ph % NBUF) * BUFR + base_a, HR)],
            outs[tower].at[pl.ds(wid * AR + quarter * HR, HR)])

    gd = [None] * NWORK
    sd = [None] * NWORK
    waited = [False] * NWORK
    for k in range(NWORK + LAG):
        if k < NWORK:
            if k >= NB and not waited[k - NB]:
                sd[k - NB].wait()
                waited[k - NB] = True
            gd[k] = gather(k)
        ks = k - LAG
        if 0 <= ks < NWORK:
            if ks % CPH == 0 and ks > 0:
                p = ks // CPH
                for i in range(ks):
                    if not waited[i]:
                        sd[i].wait()
                        waited[i] = True
                if p >= 2:
                    flush(p - 2)
                if p + 1 < NPHASES:
                    pltpu.sync_copy(
                        zeros_hbm,
                        acc_sh.at[pl.ds(((p + 1) % NBUF) * BUFR + base_a,
                                        HR)])
            tower, j = divmod(ks, NCHUNK)
            gd[ks].wait()
            if ks > 0 and not waited[ks - 1]:
                sd[ks - 1].wait()
                waited[ks - 1] = True
            sd[ks] = pltpu.async_copy(
                rows_v.at[ks % NB], acc_sh.at[sidxs[tower].at[j]], ssem,
                add=True)
    for i in range(NWORK):
        if not waited[i]:
            sd[i].wait()
    flush(NPHASES - 2)
    flush(NPHASES - 1)


def _pooled_sc(tq2, tc2, qpidx, cpidx, sidx, zeros):
    mesh = plsc.VectorSubcoreMesh(core_axis_name="c", subcore_axis_name="s")
    return pl.kernel(
        _pool_body,
        out_type=(
            jax.ShapeDtypeStruct((B * PACK, PACK * D), jnp.float32),
            jax.ShapeDtypeStruct((B * PACK, PACK * D), jnp.float32),
        ),
        mesh=mesh,
        scratch_types=[
            pltpu.VMEM((2, RPW), jnp.int32),
            pltpu.VMEM((NCHUNK, CH), jnp.int32),
            pltpu.VMEM((NCHUNK, CH), jnp.int32),
            pltpu.VMEM((NB, CH, PACK * D), jnp.float32),
            pltpu.VMEM_SHARED((NBUF * BUFR, PACK * D), jnp.float32),
            pltpu.SemaphoreType.DMA,
            pltpu.SemaphoreType.DMA,
        ],
        compiler_params=pltpu.CompilerParams(use_tc_tiling_on_sc=True),
    )(tq2, tc2, qpidx, cpidx, sidx, zeros)


def _mlp_body(aq_ref, ac_ref, wq1_ref, bq1_ref, wq2_ref, bq2_ref,
              wc1_ref, bc1_ref, wc2_ref, bc2_ref, oq_ref, oc_ref):
    blk = aq_ref.shape[0] // PACK

    def pool(a_ref):
        a = a_ref[...].reshape(blk, PACK, PACK * D)
        x = a[:, 0, 0:D]
        for r in range(1, PACK):
            x = x + a[:, r, D * r: D * (r + 1)]
        return x

    xq = pool(aq_ref)
    hq = jnp.dot(xq, wq1_ref[...], preferred_element_type=jnp.float32)
    hq = jnp.maximum(hq + bq1_ref[...], 0.0)
    oq = jnp.dot(hq, wq2_ref[...], preferred_element_type=jnp.float32)
    oq_ref[...] = jnp.maximum(oq + bq2_ref[...], 0.0)

    xc = pool(ac_ref)
    hc = jnp.dot(xc, wc1_ref[...], preferred_element_type=jnp.float32)
    hc = jnp.maximum(hc + bc1_ref[...], 0.0)
    oc = jnp.dot(hc, wc2_ref[...], preferred_element_type=jnp.float32)
    oc_ref[...] = jnp.maximum(oc + bc2_ref[...], 0.0)


def _mlp_tc(accq, accc, Wq1, bq1, Wq2, bq2, Wc1, bc1, Wc2, bc2):
    BLK = 512
    grid = (B // BLK,)
    full = lambda r, c: pl.BlockSpec((r, c), lambda i: (0, 0))
    return pl.pallas_call(
        _mlp_body,
        grid=grid,
        in_specs=[
            pl.BlockSpec((BLK * PACK, PACK * D), lambda i: (i, 0)),
            pl.BlockSpec((BLK * PACK, PACK * D), lambda i: (i, 0)),
            full(D, H1), full(1, H1), full(H1, H2), full(1, H2),
            full(D, H1), full(1, H1), full(H1, H2), full(1, H2),
        ],
        out_specs=[
            pl.BlockSpec((BLK, H2), lambda i: (i, 0)),
            pl.BlockSpec((BLK, H2), lambda i: (i, 0)),
        ],
        out_shape=[
            jax.ShapeDtypeStruct((B, H2), jnp.float32),
            jax.ShapeDtypeStruct((B, H2), jnp.float32),
        ],
    )(accq, accc, Wq1, bq1, Wq2, bq2, Wc1, bc1, Wc2, bc2)


def kernel(query_indices, candidate_indices, table_q, table_c,
           Wq1, bq1, Wq2, bq2, Wc1, bc1, Wc2, bc2):
    qidx_flat = query_indices.astype(jnp.int32).reshape(B * L)
    cidx_flat = candidate_indices.astype(jnp.int32).reshape(B * L)

    erow = jnp.arange(2 * PACK * D, dtype=jnp.int32) % (PACK * D)
    emat = (erow[:, None]
            == jnp.arange(PACK * D, dtype=jnp.int32)[None, :]).astype(
                jnp.float32)
    tq2, tc2 = _repack_tc(table_q.T, table_c.T, emat)

    qpidx = (qidx_flat // TLANE) * TROW + qidx_flat % TROW
    cpidx = (cidx_flat // TLANE) * TROW + cidx_flat % TROW
    pos = jnp.arange(B * L, dtype=jnp.int32)
    chunk_in_tower = (pos % RPW) // CH
    base = PACK * ((pos // L) % SPH) + (pos // RPW // NC) * HR
    ph_q = chunk_in_tower // CPH
    ph_c = NPH + ph_q
    sidx = jnp.stack([
        (base + (ph_q % NBUF) * BUFR
         + (qidx_flat % TLANE) // TROW).reshape(NW, NCHUNK, CH),
        (base + (ph_c % NBUF) * BUFR
         + (cidx_flat % TLANE) // TROW).reshape(NW, NCHUNK, CH),
    ]).reshape(2 * NW, NCHUNK, CH)
    zeros = jnp.zeros((HR, PACK * D), jnp.float32)

    accq, accc = _pooled_sc(tq2, tc2, qpidx, cpidx, sidx, zeros)

    q, c = _mlp_tc(accq, accc,
                   Wq1, bq1[None, :], Wq2, bq2[None, :],
                   Wc1, bc1[None, :], Wc2, bc2[None, :])
    return q, c


# speedup vs baseline: 3.0055x; 1.2069x over previous
import jax
import jax.numpy as jnp
from jax import lax
from jax.experimental import pallas as pl
from jax.experimental.pallas import tpu as pltpu
from jax.experimental.pallas import tpu_sc as plsc

B = 4096
L = 20
V = 1000000
D = 32
H1 = 128
H2 = 64

NC = 2
NS = 16
NW = NC * NS
SPW = B // NW
RPW = SPW * L
CH = 128
NCHUNK = RPW // CH
NB = 4
LAG = 2
PACK = 128 // D
AR = SPW * PACK
NPH = 4
SPH = SPW // NPH
HR = AR // NPH
CPH = NCHUNK // NPH
NBUF = 3
BUFR = NS * HR

TLANE = 8192
TROW = TLANE // PACK
TGRID = -(-V // TLANE)
VP = TGRID * TROW


def _repack_body(xq_ref, xc_ref, e_ref, oq_ref, oc_ref):
    def pack(x_ref):
        xa = x_ref[...]
        hi = xa.astype(jnp.bfloat16).astype(jnp.float32)
        lo = xa - hi
        X = jnp.concatenate(
            [hi[:, a * TROW:(a + 1) * TROW] for a in range(PACK)]
            + [lo[:, a * TROW:(a + 1) * TROW] for a in range(PACK)], axis=0)
        return lax.dot_general(X, e_ref[...], (((0,), (0,)), ((), ())),
                               preferred_element_type=jnp.float32)

    oq_ref[...] = pack(xq_ref)
    oc_ref[...] = pack(xc_ref)


def _repack_tc(tqT, tcT, emat):
    return pl.pallas_call(
        _repack_body,
        grid=(TGRID,),
        in_specs=[
            pl.BlockSpec((D, TLANE), lambda i: (0, i)),
            pl.BlockSpec((D, TLANE), lambda i: (0, i)),
            pl.BlockSpec((2 * PACK * D, PACK * D), lambda i: (0, 0)),
        ],
        out_specs=[
            pl.BlockSpec((TROW, PACK * D), lambda i: (i, 0)),
            pl.BlockSpec((TROW, PACK * D), lambda i: (i, 0)),
        ],
        out_shape=[
            jax.ShapeDtypeStruct((VP, PACK * D), jnp.float32),
            jax.ShapeDtypeStruct((VP, PACK * D), jnp.float32),
        ],
    )(tqT, tcT, emat)


def _pool_body(tq_hbm, tc_hbm, qpidx_hbm, cpidx_hbm, sidx_hbm, zeros_hbm,
               accq_hbm, accc_hbm,
               pidx_v, sidxq_v, sidxc_v, rows_v, acc_sh, gsem, ssem):
    cid = lax.axis_index("c")
    sid = lax.axis_index("s")
    wid = sid * NC + cid
    base_i = wid * RPW
    base_a = sid * HR

    pltpu.sync_copy(qpidx_hbm.at[pl.ds(base_i, RPW)], pidx_v.at[0])
    pltpu.sync_copy(cpidx_hbm.at[pl.ds(base_i, RPW)], pidx_v.at[1])
    pltpu.sync_copy(sidx_hbm.at[wid], sidxq_v)
    pltpu.sync_copy(sidx_hbm.at[NW + wid], sidxc_v)
    pltpu.sync_copy(zeros_hbm, acc_sh.at[pl.ds(0 * BUFR + base_a, HR)])
    pltpu.sync_copy(zeros_hbm, acc_sh.at[pl.ds(1 * BUFR + base_a, HR)])

    tbls = (tq_hbm, tc_hbm)
    sidxs = (sidxq_v, sidxc_v)
    outs = (accq_hbm, accc_hbm)
    NWORK = 2 * NCHUNK

    def gather(k):
        tower, j = divmod(k, NCHUNK)
        return pltpu.async_copy(
            tbls[tower].at[pidx_v.at[tower, pl.ds(j * CH, CH)]],
            rows_v.at[k % NB], gsem)

    NPHASES = 2 * NPH

    def flush(ph):
        tower, quarter = divmod(ph, NPH)
        pltpu.sync_copy(
            acc_sh.at[pl.ds((ph % NBUF) * BUFR + base_a, HR)],
            outs[tower].at[pl.ds(wid * AR + quarter * HR, HR)])

    gd = [None] * NWORK
    sd = [None] * NWORK
    waited = [False] * NWORK
    for k in range(NWORK + LAG):
        if k < NWORK:
            if k >= NB and not waited[k - NB]:
                sd[k - NB].wait()
                waited[k - NB] = True
            gd[k] = gather(k)
        ks = k - LAG
        if 0 <= ks < NWORK:
            if ks % CPH == 0 and ks > 0:
                p = ks // CPH
                for i in range(ks):
                    if not waited[i]:
                        sd[i].wait()
                        waited[i] = True
                if p >= 2:
                    flush(p - 2)
                if p + 1 < NPHASES:
                    pltpu.sync_copy(
                        zeros_hbm,
                        acc_sh.at[pl.ds(((p + 1) % NBUF) * BUFR + base_a,
                                        HR)])
            tower, j = divmod(ks, NCHUNK)
            gd[ks].wait()
            if ks > 0 and not waited[ks - 1]:
                sd[ks - 1].wait()
                waited[ks - 1] = True
            sd[ks] = pltpu.async_copy(
                rows_v.at[ks % NB], acc_sh.at[sidxs[tower].at[j]], ssem,
                add=True)
    for i in range(NWORK):
        if not waited[i]:
            sd[i].wait()
    flush(NPHASES - 2)
    flush(NPHASES - 1)


def _pooled_sc(tq2, tc2, qpidx, cpidx, sidx, zeros):
    mesh = plsc.VectorSubcoreMesh(core_axis_name="c", subcore_axis_name="s")
    return pl.kernel(
        _pool_body,
        out_type=(
            jax.ShapeDtypeStruct((B * PACK, PACK * D), jnp.float32),
            jax.ShapeDtypeStruct((B * PACK, PACK * D), jnp.float32),
        ),
        mesh=mesh,
        scratch_types=[
            pltpu.VMEM((2, RPW), jnp.int32),
            pltpu.VMEM((NCHUNK, CH), jnp.int32),
            pltpu.VMEM((NCHUNK, CH), jnp.int32),
            pltpu.VMEM((NB, CH, PACK * D), jnp.float32),
            pltpu.VMEM_SHARED((NBUF * BUFR, PACK * D), jnp.float32),
            pltpu.SemaphoreType.DMA,
            pltpu.SemaphoreType.DMA,
        ],
        compiler_params=pltpu.CompilerParams(use_tc_tiling_on_sc=True),
    )(tq2, tc2, qpidx, cpidx, sidx, zeros)


def _mlp_body(aq_ref, ac_ref, wq1_ref, bq1_ref, wq2_ref, bq2_ref,
              wc1_ref, bc1_ref, wc2_ref, bc2_ref, oq_ref, oc_ref):
    blk = aq_ref.shape[0] // PACK

    def pool(a_ref):
        a = a_ref[...].reshape(blk, PACK, PACK * D)
        x = a[:, 0, 0:D]
        for r in range(1, PACK):
            x = x + a[:, r, D * r: D * (r + 1)]
        return x

    xq = pool(aq_ref)
    hq = jnp.dot(xq, wq1_ref[...], preferred_element_type=jnp.float32)
    hq = jnp.maximum(hq + bq1_ref[...], 0.0)
    oq = jnp.dot(hq, wq2_ref[...], preferred_element_type=jnp.float32)
    oq_ref[...] = jnp.maximum(oq + bq2_ref[...], 0.0)

    xc = pool(ac_ref)
    hc = jnp.dot(xc, wc1_ref[...], preferred_element_type=jnp.float32)
    hc = jnp.maximum(hc + bc1_ref[...], 0.0)
    oc = jnp.dot(hc, wc2_ref[...], preferred_element_type=jnp.float32)
    oc_ref[...] = jnp.maximum(oc + bc2_ref[...], 0.0)


def _mlp_tc(accq, accc, Wq1, bq1, Wq2, bq2, Wc1, bc1, Wc2, bc2):
    BLK = 512
    grid = (B // BLK,)
    full = lambda r, c: pl.BlockSpec((r, c), lambda i: (0, 0))
    return pl.pallas_call(
        _mlp_body,
        grid=grid,
        in_specs=[
            pl.BlockSpec((BLK * PACK, PACK * D), lambda i: (i, 0)),
            pl.BlockSpec((BLK * PACK, PACK * D), lambda i: (i, 0)),
            full(D, H1), full(1, H1), full(H1, H2), full(1, H2),
            full(D, H1), full(1, H1), full(H1, H2), full(1, H2),
        ],
        out_specs=[
            pl.BlockSpec((BLK, H2), lambda i: (i, 0)),
            pl.BlockSpec((BLK, H2), lambda i: (i, 0)),
        ],
        out_shape=[
            jax.ShapeDtypeStruct((B, H2), jnp.float32),
            jax.ShapeDtypeStruct((B, H2), jnp.float32),
        ],
    )(accq, accc, Wq1, bq1, Wq2, bq2, Wc1, bc1, Wc2, bc2)


def kernel(query_indices, candidate_indices, table_q, table_c,
           Wq1, bq1, Wq2, bq2, Wc1, bc1, Wc2, bc2):
    qidx_flat = query_indices.astype(jnp.int32).reshape(B * L)
    cidx_flat = candidate_indices.astype(jnp.int32).reshape(B * L)

    erow = jnp.arange(2 * PACK * D, dtype=jnp.int32) % (PACK * D)
    emat = (erow[:, None]
            == jnp.arange(PACK * D, dtype=jnp.int32)[None, :]).astype(
                jnp.float32)
    tq2, tc2 = _repack_tc(table_q.T, table_c.T, emat)

    qpidx = (qidx_flat // TLANE) * TROW + qidx_flat % TROW
    cpidx = (cidx_flat // TLANE) * TROW + cidx_flat % TROW
    pos = jnp.arange(B * L, dtype=jnp.int32)
    chunk_in_tower = (pos % RPW) // CH
    base = PACK * ((pos // L) % SPH) + (pos // RPW // NC) * HR
    ph_q = chunk_in_tower // CPH
    ph_c = NPH + ph_q
    sidx = jnp.stack([
        (base + (ph_q % NBUF) * BUFR
         + (qidx_flat % TLANE) // TROW).reshape(NW, NCHUNK, CH),
        (base + (ph_c % NBUF) * BUFR
         + (cidx_flat % TLANE) // TROW).reshape(NW, NCHUNK, CH),
    ]).reshape(2 * NW, NCHUNK, CH)
    zeros = jnp.zeros((HR, PACK * D), jnp.float32)

    accq, accc = _pooled_sc(tq2, tc2, qpidx, cpidx, sidx, zeros)

    q, c = _mlp_tc(accq, accc,
                   Wq1, bq1[None, :], Wq2, bq2[None, :],
                   Wc1, bc1[None, :], Wc2, bc2[None, :])
    return q, c


# speedup vs baseline: 3.3875x; 1.1271x over previous
import jax
import jax.numpy as jnp
from jax import lax
from jax.experimental import pallas as pl
from jax.experimental.pallas import tpu as pltpu
from jax.experimental.pallas import tpu_sc as plsc

B = 4096
L = 20
V = 1000000
D = 32
H1 = 128
H2 = 64

NC = 2
NS = 16
NW = NC * NS
SPW = B // NW
RPW = SPW * L
CH = 128
NCHUNK = RPW // CH
NB = 4
LAG = 2
PACK = 128 // D
AR = SPW * PACK
NPH = 4
SPH = SPW // NPH
HR = AR // NPH
CPH = NCHUNK // NPH
NBUF = 3
BUFR = NS * HR

TLANE = 16384
TROW = TLANE // PACK
TGRID = -(-V // TLANE)
VP = TGRID * TROW


def _repack_body(xq_ref, xc_ref, e_ref, oq_ref, oc_ref):
    def pack(x_ref):
        xa = x_ref[...]
        hi = xa.astype(jnp.bfloat16).astype(jnp.float32)
        lo = xa - hi
        X = jnp.concatenate(
            [hi[:, a * TROW:(a + 1) * TROW] for a in range(PACK)]
            + [lo[:, a * TROW:(a + 1) * TROW] for a in range(PACK)], axis=0)
        return lax.dot_general(X, e_ref[...], (((0,), (0,)), ((), ())),
                               preferred_element_type=jnp.float32)

    oq_ref[...] = pack(xq_ref)
    oc_ref[...] = pack(xc_ref)


def _repack_tc(tqT, tcT, emat):
    return pl.pallas_call(
        _repack_body,
        grid=(TGRID,),
        in_specs=[
            pl.BlockSpec((D, TLANE), lambda i: (0, i)),
            pl.BlockSpec((D, TLANE), lambda i: (0, i)),
            pl.BlockSpec((2 * PACK * D, PACK * D), lambda i: (0, 0)),
        ],
        out_specs=[
            pl.BlockSpec((TROW, PACK * D), lambda i: (i, 0)),
            pl.BlockSpec((TROW, PACK * D), lambda i: (i, 0)),
        ],
        out_shape=[
            jax.ShapeDtypeStruct((VP, PACK * D), jnp.float32),
            jax.ShapeDtypeStruct((VP, PACK * D), jnp.float32),
        ],
    )(tqT, tcT, emat)


def _pool_body(tq_hbm, tc_hbm, qpidx_hbm, cpidx_hbm, sidx_hbm, zeros_hbm,
               accq_hbm, accc_hbm,
               pidx_v, sidxq_v, sidxc_v, rows_v, acc_sh, gsem, ssem):
    cid = lax.axis_index("c")
    sid = lax.axis_index("s")
    wid = sid * NC + cid
    base_i = wid * RPW
    base_a = sid * HR

    pltpu.sync_copy(qpidx_hbm.at[pl.ds(base_i, RPW)], pidx_v.at[0])
    pltpu.sync_copy(cpidx_hbm.at[pl.ds(base_i, RPW)], pidx_v.at[1])
    pltpu.sync_copy(sidx_hbm.at[wid], sidxq_v)
    pltpu.sync_copy(sidx_hbm.at[NW + wid], sidxc_v)
    pltpu.sync_copy(zeros_hbm, acc_sh.at[pl.ds(0 * BUFR + base_a, HR)])
    pltpu.sync_copy(zeros_hbm, acc_sh.at[pl.ds(1 * BUFR + base_a, HR)])

    tbls = (tq_hbm, tc_hbm)
    sidxs = (sidxq_v, sidxc_v)
    outs = (accq_hbm, accc_hbm)
    NWORK = 2 * NCHUNK

    def gather(k):
        tower, j = divmod(k, NCHUNK)
        return pltpu.async_copy(
            tbls[tower].at[pidx_v.at[tower, pl.ds(j * CH, CH)]],
            rows_v.at[k % NB], gsem)

    NPHASES = 2 * NPH

    def flush(ph):
        tower, quarter = divmod(ph, NPH)
        pltpu.sync_copy(
            acc_sh.at[pl.ds((ph % NBUF) * BUFR + base_a, HR)],
            outs[tower].at[pl.ds(wid * AR + quarter * HR, HR)])

    gd = [None] * NWORK
    sd = [None] * NWORK
    waited = [False] * NWORK
    for k in range(NWORK + LAG):
        if k < NWORK:
            if k >= NB and not waited[k - NB]:
                sd[k - NB].wait()
                waited[k - NB] = True
            gd[k] = gather(k)
        ks = k - LAG
        if 0 <= ks < NWORK:
            if ks % CPH == 0 and ks > 0:
                p = ks // CPH
                for i in range(ks):
                    if not waited[i]:
                        sd[i].wait()
                        waited[i] = True
                if p >= 2:
                    flush(p - 2)
                if p + 1 < NPHASES:
                    pltpu.sync_copy(
                        zeros_hbm,
                        acc_sh.at[pl.ds(((p + 1) % NBUF) * BUFR + base_a,
                                        HR)])
            tower, j = divmod(ks, NCHUNK)
            gd[ks].wait()
            if ks > 0 and not waited[ks - 1]:
                sd[ks - 1].wait()
                waited[ks - 1] = True
            sd[ks] = pltpu.async_copy(
                rows_v.at[ks % NB], acc_sh.at[sidxs[tower].at[j]], ssem,
                add=True)
    for i in range(NWORK):
        if not waited[i]:
            sd[i].wait()
    flush(NPHASES - 2)
    flush(NPHASES - 1)


def _pooled_sc(tq2, tc2, qpidx, cpidx, sidx, zeros):
    mesh = plsc.VectorSubcoreMesh(core_axis_name="c", subcore_axis_name="s")
    return pl.kernel(
        _pool_body,
        out_type=(
            jax.ShapeDtypeStruct((B * PACK, PACK * D), jnp.float32),
            jax.ShapeDtypeStruct((B * PACK, PACK * D), jnp.float32),
        ),
        mesh=mesh,
        scratch_types=[
            pltpu.VMEM((2, RPW), jnp.int32),
            pltpu.VMEM((NCHUNK, CH), jnp.int32),
            pltpu.VMEM((NCHUNK, CH), jnp.int32),
            pltpu.VMEM((NB, CH, PACK * D), jnp.float32),
            pltpu.VMEM_SHARED((NBUF * BUFR, PACK * D), jnp.float32),
            pltpu.SemaphoreType.DMA,
            pltpu.SemaphoreType.DMA,
        ],
        compiler_params=pltpu.CompilerParams(use_tc_tiling_on_sc=True),
    )(tq2, tc2, qpidx, cpidx, sidx, zeros)


def _mlp_body(aq_ref, ac_ref, wq1_ref, bq1_ref, wq2_ref, bq2_ref,
              wc1_ref, bc1_ref, wc2_ref, bc2_ref, oq_ref, oc_ref):
    blk = aq_ref.shape[0] // PACK

    def pool(a_ref):
        a = a_ref[...].reshape(blk, PACK, PACK * D)
        x = a[:, 0, 0:D]
        for r in range(1, PACK):
            x = x + a[:, r, D * r: D * (r + 1)]
        return x

    xq = pool(aq_ref)
    hq = jnp.dot(xq, wq1_ref[...], preferred_element_type=jnp.float32)
    hq = jnp.maximum(hq + bq1_ref[...], 0.0)
    oq = jnp.dot(hq, wq2_ref[...], preferred_element_type=jnp.float32)
    oq_ref[...] = jnp.maximum(oq + bq2_ref[...], 0.0)

    xc = pool(ac_ref)
    hc = jnp.dot(xc, wc1_ref[...], preferred_element_type=jnp.float32)
    hc = jnp.maximum(hc + bc1_ref[...], 0.0)
    oc = jnp.dot(hc, wc2_ref[...], preferred_element_type=jnp.float32)
    oc_ref[...] = jnp.maximum(oc + bc2_ref[...], 0.0)


def _mlp_tc(accq, accc, Wq1, bq1, Wq2, bq2, Wc1, bc1, Wc2, bc2):
    BLK = 512
    grid = (B // BLK,)
    full = lambda r, c: pl.BlockSpec((r, c), lambda i: (0, 0))
    return pl.pallas_call(
        _mlp_body,
        grid=grid,
        in_specs=[
            pl.BlockSpec((BLK * PACK, PACK * D), lambda i: (i, 0)),
            pl.BlockSpec((BLK * PACK, PACK * D), lambda i: (i, 0)),
            full(D, H1), full(1, H1), full(H1, H2), full(1, H2),
            full(D, H1), full(1, H1), full(H1, H2), full(1, H2),
        ],
        out_specs=[
            pl.BlockSpec((BLK, H2), lambda i: (i, 0)),
            pl.BlockSpec((BLK, H2), lambda i: (i, 0)),
        ],
        out_shape=[
            jax.ShapeDtypeStruct((B, H2), jnp.float32),
            jax.ShapeDtypeStruct((B, H2), jnp.float32),
        ],
    )(accq, accc, Wq1, bq1, Wq2, bq2, Wc1, bc1, Wc2, bc2)


def kernel(query_indices, candidate_indices, table_q, table_c,
           Wq1, bq1, Wq2, bq2, Wc1, bc1, Wc2, bc2):
    qidx_flat = query_indices.astype(jnp.int32).reshape(B * L)
    cidx_flat = candidate_indices.astype(jnp.int32).reshape(B * L)

    erow = jnp.arange(2 * PACK * D, dtype=jnp.int32) % (PACK * D)
    emat = (erow[:, None]
            == jnp.arange(PACK * D, dtype=jnp.int32)[None, :]).astype(
                jnp.float32)
    tq2, tc2 = _repack_tc(table_q.T, table_c.T, emat)

    qpidx = (qidx_flat // TLANE) * TROW + qidx_flat % TROW
    cpidx = (cidx_flat // TLANE) * TROW + cidx_flat % TROW
    pos = jnp.arange(B * L, dtype=jnp.int32)
    chunk_in_tower = (pos % RPW) // CH
    base = PACK * ((pos // L) % SPH) + (pos // RPW // NC) * HR
    ph_q = chunk_in_tower // CPH
    ph_c = NPH + ph_q
    sidx = jnp.stack([
        (base + (ph_q % NBUF) * BUFR
         + (qidx_flat % TLANE) // TROW).reshape(NW, NCHUNK, CH),
        (base + (ph_c % NBUF) * BUFR
         + (cidx_flat % TLANE) // TROW).reshape(NW, NCHUNK, CH),
    ]).reshape(2 * NW, NCHUNK, CH)
    zeros = jnp.zeros((HR, PACK * D), jnp.float32)

    accq, accc = _pooled_sc(tq2, tc2, qpidx, cpidx, sidx, zeros)

    q, c = _mlp_tc(accq, accc,
                   Wq1, bq1[None, :], Wq2, bq2[None, :],
                   Wc1, bc1[None, :], Wc2, bc2[None, :])
    return q, c
